# baseline (speedup 1.0000x reference)
import functools

import jax
import jax.numpy as jnp
from jax import lax
from jax.experimental import pallas as pl
from jax.experimental.pallas import tpu as pltpu
from jax.experimental.pallas import tpu_sc as plsc


def _prep_block(xT_ref, es_ref, cu_ref, cb_ref, cm2_ref, c2_ref, x2_ref,
                *, K, TM):
    cb = es_ref[...] / jnp.maximum(cu_ref[...], 1e-8)
    cb_ref[...] = jnp.concatenate(
        [cb, jnp.zeros((K, 128 - cb.shape[1]), jnp.float32)], axis=1)
    cm2_ref[...] = -2.0 * cb
    c2 = jnp.sum(cb * cb, axis=1, keepdims=True)
    c2_ref[...] = jnp.broadcast_to(c2, (K, TM))
    x2_ref[...] = jnp.sum(xT_ref[...] * xT_ref[...], axis=0, keepdims=True)


def _prep(xT, es, cu2, N, D, K, TM):
    return pl.pallas_call(
        functools.partial(_prep_block, K=K, TM=TM),
        grid=(1,),
        in_specs=[
            pl.BlockSpec((D, N), lambda i: (0, 0)),
            pl.BlockSpec((K, D), lambda i: (0, 0)),
            pl.BlockSpec((K, 1), lambda i: (0, 0)),
        ],
        out_specs=[
            pl.BlockSpec((K, 128), lambda i: (0, 0)),
            pl.BlockSpec((K, D), lambda i: (0, 0)),
            pl.BlockSpec((K, TM), lambda i: (0, 0)),
            pl.BlockSpec((1, N), lambda i: (0, 0)),
        ],
        out_shape=[
            jax.ShapeDtypeStruct((K, 128), jnp.float32),
            jax.ShapeDtypeStruct((K, D), jnp.float32),
            jax.ShapeDtypeStruct((K, TM), jnp.float32),
            jax.ShapeDtypeStruct((1, N), jnp.float32),
        ],
    )(xT, es, cu2)


def _encode_block(xT_ref, cm2_ref, c2_ref, x2_ref, idx_ref, loss_ref,
                  *, K, TM, NT, inv_count):
    i = pl.program_id(0)
    xT = xT_ref[...]
    sT = lax.dot_general(cm2_ref[...], xT, (((1,), (0,)), ((), ())),
                         preferred_element_type=jnp.float32)
    d2 = x2_ref[...] + sT + c2_ref[...]
    md = jnp.min(d2, axis=0, keepdims=True)
    c = jnp.maximum(md, 0.0)
    s = jnp.sqrt(c)
    base = s * s
    bi = lax.bitcast_convert_type(base, jnp.int32)
    hi = md
    for k in range(5):
        hk = lax.bitcast_convert_type(bi + k, jnp.float32)
        ok = jnp.sqrt(jnp.maximum(hk, 0.0)) == s
        hi = jnp.where(ok, jnp.maximum(hi, hk), hi)
    ids = lax.broadcasted_iota(jnp.int32, (K, TM), 0)
    idx_ref[...] = jnp.min(jnp.where(d2 <= hi, ids, K), axis=0, keepdims=True)
    bs = jnp.sum(c, keepdims=True).reshape(1, 1)
    prev = jnp.where(i == 0, jnp.zeros((1, 1), jnp.float32), loss_ref[...])
    tot = prev + bs
    loss_ref[...] = jnp.where(i == NT - 1, tot * inv_count, tot)


def _encode(xT, cm2, c2b, x2, N, D, K, TM):
    NT = N // TM
    body = functools.partial(_encode_block, K=K, TM=TM, NT=NT,
                             inv_count=1.0 / (N * D))
    return pl.pallas_call(
        body,
        grid=(NT,),
        in_specs=[
            pl.BlockSpec((D, TM), lambda i: (0, i)),
            pl.BlockSpec((K, D), lambda i: (0, 0)),
            pl.BlockSpec((K, TM), lambda i: (0, 0)),
            pl.BlockSpec((1, TM), lambda i: (0, i)),
        ],
        out_specs=[
            pl.BlockSpec((1, TM), lambda i: (0, i)),
            pl.BlockSpec((1, 1), lambda i: (0, 0)),
        ],
        out_shape=[
            jax.ShapeDtypeStruct((1, N), jnp.int32),
            jax.ShapeDtypeStruct((1, 1), jnp.float32),
        ],
    )(xT, cm2, c2b, x2)


def _sc_gather(cb_p, idx2d, N, D):
    Dp = cb_p.shape[1]
    rows_per_w = idx2d.shape[0] // 32
    b_per_w = rows_per_w * 128
    mesh = plsc.VectorSubcoreMesh(core_axis_name="c", subcore_axis_name="s")

    @functools.partial(
        pl.kernel, mesh=mesh,
        out_type=jax.ShapeDtypeStruct((N, Dp), jnp.float32),
        scratch_types=[
            pltpu.VMEM((rows_per_w, 128), jnp.int32),
            pltpu.VMEM((b_per_w, Dp), jnp.float32),
            pltpu.SemaphoreType.DMA,
        ],
    )
    def k(cb_hbm, idx_hbm, out_hbm, idx_v, rows_v, sem):
        wid = lax.axis_index("s") * 2 + lax.axis_index("c")
        pltpu.sync_copy(idx_hbm.at[pl.ds(wid * rows_per_w, rows_per_w)], idx_v)
        copies = []
        for j in range(rows_per_w):
            copies.append(pltpu.async_copy(
                cb_hbm.at[idx_v.at[j]],
                rows_v.at[pl.ds(j * 128, 128)], sem))
        for c in copies:
            c.wait()
        pltpu.sync_copy(rows_v, out_hbm.at[pl.ds(wid * b_per_w, b_per_w)])

    return k(cb_p, idx2d)


def kernel(x, embedding_sum, cluster_usage):
    B, T, D = x.shape
    N = B * T
    K = embedding_sum.shape[0]
    TM = 256

    flat = x.astype(jnp.float32).reshape(N, D)
    xT = flat.T
    cu2 = cluster_usage.astype(jnp.float32).reshape(K, 1)
    es = embedding_sum.astype(jnp.float32)

    cb_p, cm2, c2b, x2 = _prep(xT, es, cu2, N, D, K, TM)
    idx_row, loss11 = _encode(xT, cm2, c2b, x2, N, D, K, TM)
    idx_flat = idx_row.reshape(N)
    q = _sc_gather(cb_p, idx_flat.reshape(N // 128, 128), N, D)

    out = q[:, :D].reshape(x.shape)
    indices = idx_flat.reshape(B, T)
    commitment_loss = loss11[0, 0]
    return (out, indices, commitment_loss)


# speedup vs baseline: 1.0761x; 1.0761x over previous
import functools

import jax
import jax.numpy as jnp
from jax import lax
from jax.experimental import pallas as pl
from jax.experimental.pallas import tpu as pltpu
from jax.experimental.pallas import tpu_sc as plsc


def _encode_block(xT_ref, es_ref, cu_ref, idx_ref, loss_ref, cb_ref,
                  c2_ref, cm2_ref, ids_ref, *, K, TM, NT, inv_count):
    i = pl.program_id(0)

    @pl.when(i == 0)
    def _init():
        cb = es_ref[...] / jnp.maximum(cu_ref[...], 1e-8)
        cb_ref[...] = jnp.concatenate(
            [cb, jnp.zeros((K, 128 - cb.shape[1]), jnp.float32)], axis=1)
        cm2_ref[...] = -2.0 * cb
        c2 = jnp.sum(cb * cb, axis=1, keepdims=True)
        c2_ref[...] = jnp.broadcast_to(c2, (K, TM))
        ids_ref[...] = lax.broadcasted_iota(
            jnp.int32, (K, TM), 0).astype(jnp.float32)

    xT = xT_ref[...]
    sT = lax.dot_general(cm2_ref[...], xT, (((1,), (0,)), ((), ())),
                         preferred_element_type=jnp.float32)
    x2 = jnp.sum(xT * xT, axis=0, keepdims=True)
    d2 = x2 + sT + c2_ref[...]
    md = jnp.min(d2, axis=0, keepdims=True)
    c = jnp.maximum(md, 0.0)
    s = jnp.sqrt(c)
    base = s * s
    bi = lax.bitcast_convert_type(base, jnp.int32)
    hi = md
    for k in range(5):
        hk = lax.bitcast_convert_type(bi + k, jnp.float32)
        ok = jnp.sqrt(jnp.maximum(hk, 0.0)) == s
        hi = jnp.where(ok, jnp.maximum(hi, hk), hi)
    idxf = jnp.min(jnp.where(d2 <= hi, ids_ref[...], jnp.float32(K)),
                   axis=0, keepdims=True)
    idx_ref[...] = idxf.astype(jnp.int32)
    bs = jnp.sum(c, keepdims=True).reshape(1, 1)
    prev = jnp.where(i == 0, jnp.zeros((1, 1), jnp.float32), loss_ref[...])
    tot = prev + bs
    loss_ref[...] = jnp.where(i == NT - 1, tot * inv_count, tot)


def _encode(xT, es, cu2, N, D, K, TM):
    NT = N // TM
    body = functools.partial(_encode_block, K=K, TM=TM, NT=NT,
                             inv_count=1.0 / (N * D))
    return pl.pallas_call(
        body,
        grid=(NT,),
        in_specs=[
            pl.BlockSpec((D, TM), lambda i: (0, i)),
            pl.BlockSpec((K, D), lambda i: (0, 0)),
            pl.BlockSpec((K, 1), lambda i: (0, 0)),
        ],
        out_specs=[
            pl.BlockSpec((1, TM), lambda i: (0, i)),
            pl.BlockSpec((1, 1), lambda i: (0, 0)),
            pl.BlockSpec((K, 128), lambda i: (0, 0)),
        ],
        out_shape=[
            jax.ShapeDtypeStruct((1, N), jnp.int32),
            jax.ShapeDtypeStruct((1, 1), jnp.float32),
            jax.ShapeDtypeStruct((K, 128), jnp.float32),
        ],
        scratch_shapes=[pltpu.VMEM((K, TM), jnp.float32),
                        pltpu.VMEM((K, D), jnp.float32),
                        pltpu.VMEM((K, TM), jnp.float32)],
    )(xT, es, cu2)


def _sc_gather(cb_p, idx2d, N, D):
    Dp = cb_p.shape[1]
    rows_per_w = idx2d.shape[0] // 32
    b_per_w = rows_per_w * 128
    mesh = plsc.VectorSubcoreMesh(core_axis_name="c", subcore_axis_name="s")

    @functools.partial(
        pl.kernel, mesh=mesh,
        out_type=jax.ShapeDtypeStruct((N, Dp), jnp.float32),
        scratch_types=[
            pltpu.VMEM((rows_per_w, 128), jnp.int32),
            pltpu.VMEM((b_per_w, Dp), jnp.float32),
            pltpu.SemaphoreType.DMA,
        ],
    )
    def k(cb_hbm, idx_hbm, out_hbm, idx_v, rows_v, sem):
        wid = lax.axis_index("s") * 2 + lax.axis_index("c")
        pltpu.sync_copy(idx_hbm.at[pl.ds(wid * rows_per_w, rows_per_w)], idx_v)
        copies = []
        for j in range(rows_per_w):
            copies.append(pltpu.async_copy(
                cb_hbm.at[idx_v.at[j]],
                rows_v.at[pl.ds(j * 128, 128)], sem))
        for c in copies:
            c.wait()
        pltpu.sync_copy(rows_v, out_hbm.at[pl.ds(wid * b_per_w, b_per_w)])

    return k(cb_p, idx2d)


def kernel(x, embedding_sum, cluster_usage):
    B, T, D = x.shape
    N = B * T
    K = embedding_sum.shape[0]
    TM = 256

    flat = x.astype(jnp.float32).reshape(N, D)
    xT = flat.T
    cu2 = cluster_usage.astype(jnp.float32).reshape(K, 1)
    es = embedding_sum.astype(jnp.float32)

    idx_row, loss11, cb_p = _encode(xT, es, cu2, N, D, K, TM)
    idx_flat = idx_row.reshape(N)
    q = _sc_gather(cb_p, idx_flat.reshape(N // 128, 128), N, D)

    out = q[:, :D].reshape(x.shape)
    indices = idx_flat.reshape(B, T)
    commitment_loss = loss11[0, 0]
    return (out, indices, commitment_loss)
